# baseline (speedup 1.0000x reference)
import jax
import jax.numpy as jnp
import numpy as np
from jax import lax
from jax.experimental import pallas as pl
from jax.experimental.pallas import tpu as pltpu
from jax.experimental.pallas import tpu_sc as plsc

_VOCAB = 100000
_B = 1024
_L = 200
_BIAS = float(np.log(12000 / 10000))

_NC = 2
_NS = 16
_NW = _NC * _NS
_ROWS_PER_W = _B // _NW
_IDS_PER_W = _ROWS_PER_W * _L
_HALF = _IDS_PER_W // 2
_BLK = 128


def _sc_body(idx_hbm, r_hbm, out_hbm, r_sh, idx2_v, idx_v,
             vals0_v, vals1_v, out_v, sem0, sem1):
    sid = lax.axis_index("s")
    cid = lax.axis_index("c")
    wid = cid * _NS + sid
    row0 = wid * _ROWS_PER_W

    @pl.when(sid == 0)
    def _():
        pltpu.sync_copy(r_hbm, r_sh)

    blk0 = pl.multiple_of((wid // 4) * _BLK, _BLK)
    pltpu.sync_copy(idx_hbm.at[:, pl.ds(blk0, _BLK)], idx2_v)

    c0 = (wid % 4) * _ROWS_PER_W

    def compact_row(l, _):
        for h in (0, 16):
            v = idx2_v[l, pl.ds(c0 + h, 16)]
            v = jnp.minimum(jnp.maximum(v, 0), _VOCAB - 1)
            idx_v[pl.ds(l * _ROWS_PER_W + h, 16)] = v
        return 0

    lax.fori_loop(0, _L, compact_row, 0)

    plsc.subcore_barrier()

    cp0 = pltpu.async_copy(r_sh.at[idx_v.at[pl.ds(0, _HALF)]], vals0_v, sem0)
    cp1 = pltpu.async_copy(r_sh.at[idx_v.at[pl.ds(_HALF, _HALF)]], vals1_v, sem1)

    init = (jnp.full((16,), _BIAS, jnp.float32), jnp.zeros((16,), jnp.float32))

    def acc_half(vref, carry):
        def body(l, ab):
            a, b = ab
            off = l * _ROWS_PER_W
            return (a + vref[pl.ds(off, 16)], b + vref[pl.ds(off + 16, 16)])
        return plsc.parallel_loop(0, _L // 2, carry=carry, unroll=4)(body)

    cp0.wait()
    carry = acc_half(vals0_v, init)
    cp1.wait()
    a0, a1 = acc_half(vals1_v, carry)
    a1 = a1 + jnp.full((16,), _BIAS, jnp.float32)
    out_v[pl.ds(0, 16)] = a0
    out_v[pl.ds(16, 16)] = a1
    pltpu.sync_copy(out_v, out_hbm.at[pl.ds(row0, _ROWS_PER_W)])


@jax.jit
def _run(idx_t, r):
    mesh = plsc.VectorSubcoreMesh(core_axis_name="c", subcore_axis_name="s")
    return pl.kernel(
        _sc_body,
        mesh=mesh,
        compiler_params=pltpu.CompilerParams(needs_layout_passes=False),
        out_type=jax.ShapeDtypeStruct((_B,), jnp.float32),
        scratch_types=[
            pltpu.VMEM_SHARED((_VOCAB,), jnp.float32),
            pltpu.VMEM((_L, _BLK), jnp.int32),
            pltpu.VMEM((_IDS_PER_W,), jnp.int32),
            pltpu.VMEM((_HALF,), jnp.float32),
            pltpu.VMEM((_HALF,), jnp.float32),
            pltpu.VMEM((_ROWS_PER_W,), jnp.float32),
            pltpu.SemaphoreType.DMA,
            pltpu.SemaphoreType.DMA,
        ],
    )(idx_t, r)


def kernel(batch, r):
    return _run(batch.astype(jnp.int32).T, r)
